# baseline (speedup 1.0000x reference)
import functools

import jax
import jax.numpy as jnp
from jax import lax
from jax.experimental import pallas as pl
from jax.experimental.pallas import tpu as pltpu
from jax.experimental.pallas import tpu_sc as plsc

B = 16384
N_TOKEN_FIELDS = 26
N_SEQ_FIELDS = 2
N_FLOAT = 13
N_FLOAT_SEQ = 4
SEQ_LEN = 50
VOCAB = 100000
D = 16
NBLK = 30

NC = 2
NS = 16
NW = NC * NS
ROWS_PER_W = B // NW
NB = 32
NCHUNK = ROWS_PER_W // NB

TOK_SUB = 96
SEQ_SUB = 128


def _prep_body(tf_ref, tsf_ref, ff_ref, fsq_ref, numW_ref, numb_ref,
               nsW_ref, nsb_ref, tok_idx_ref, seq_idx_ref, dense_ref):
    bsz = tf_ref.shape[0]
    tf = tf_ref[...]
    f_ids = lax.broadcasted_iota(jnp.int32, (bsz, N_TOKEN_FIELDS), 1)
    tok_idx = tf + f_ids * VOCAB
    tok_idx_ref[...] = jnp.concatenate(
        [tok_idx, jnp.zeros((bsz, NBLK - N_TOKEN_FIELDS), jnp.int32)], axis=1)
    tsf = tsf_ref[...]
    s_ids = lax.broadcasted_iota(jnp.int32, (bsz, N_SEQ_FIELDS * SEQ_LEN), 1)
    seq_idx_ref[...] = tsf + (s_ids // SEQ_LEN) * VOCAB
    ff = ff_ref[...]
    num = jnp.broadcast_to(numb_ref[...][None, :], (bsz, D))
    for k in range(N_FLOAT):
        num = num + ff[:, k:k + 1] * numW_ref[k:k + 1, :]
    fsq = fsq_ref[...]
    ns = jnp.broadcast_to(nsb_ref[...][None, :], (bsz, D))
    for f in range(N_FLOAT_SEQ):
        seg = fsq[:, f * SEQ_LEN:(f + 1) * SEQ_LEN]
        cnt = jnp.sum((seg != 0.0).astype(jnp.float32), axis=1, keepdims=True)
        feat = jnp.sum(seg, axis=1, keepdims=True) / (cnt + 1e-08)
        ns = ns + feat * nsW_ref[f:f + 1, :]
    dense_ref[...] = jnp.concatenate([num, ns], axis=1)


def _prep(token_feature, token_seq2d, float_feature, float_seq2d,
          num_W, num_b, numseq_W, numseq_b):
    bsz = 2048
    grid = B // bsz
    row_blk = lambda w: pl.BlockSpec((bsz, w), lambda i: (i, 0))
    full = lambda shp: pl.BlockSpec(shp, lambda i: tuple(0 for _ in shp))
    return pl.pallas_call(
        _prep_body,
        grid=(grid,),
        in_specs=[
            row_blk(N_TOKEN_FIELDS),
            row_blk(N_SEQ_FIELDS * SEQ_LEN),
            row_blk(N_FLOAT),
            row_blk(N_FLOAT_SEQ * SEQ_LEN),
            full((N_FLOAT, D)),
            full((D,)),
            full((N_FLOAT_SEQ, D)),
            full((D,)),
        ],
        out_specs=[
            row_blk(NBLK),
            row_blk(N_SEQ_FIELDS * SEQ_LEN),
            row_blk(2 * D),
        ],
        out_shape=[
            jax.ShapeDtypeStruct((B, NBLK), jnp.int32),
            jax.ShapeDtypeStruct((B, N_SEQ_FIELDS * SEQ_LEN), jnp.int32),
            jax.ShapeDtypeStruct((B, 2 * D), jnp.float32),
        ],
    )(token_feature, token_seq2d, float_feature, float_seq2d,
      num_W, num_b, numseq_W, numseq_b)


def _tr_body(tabT, stabT, tok_tail, seq_tail, tok_lin, seq_lin, in_v, out_v, sem):
    wid = lax.axis_index("s") * NC + lax.axis_index("c")
    rows16 = lax.iota(jnp.int32, 16)

    def make_block(src, dst):
        def block(g, _):
            pltpu.sync_copy(src.at[:, pl.ds(g * 128, 128)], in_v)
            for l in range(128):
                lane = jnp.full((16,), l, jnp.int32)
                out_v[pl.ds(l * 16, 16)] = plsc.load_gather(in_v, [rows16, lane])
            pltpu.sync_copy(out_v, dst.at[pl.ds(g * 2048, 2048)])
            return 0
        return block

    def tail(src_lin, dst, g):
        pltpu.sync_copy(src_lin, out_v.at[pl.ds(0, 1024)])
        pltpu.sync_copy(out_v.at[pl.ds(0, 1024)], dst.at[pl.ds(g * 2048, 1024)])

    nb_tok = (VOCAB * N_TOKEN_FIELDS) // 128
    nb_seq = (VOCAB * N_SEQ_FIELDS) // 128
    per_w = nb_tok // NW
    blk_tok = make_block(tabT, tok_lin)
    lax.fori_loop(wid * per_w, (wid + 1) * per_w, blk_tok, 0)

    @pl.when(wid < nb_tok - per_w * NW)
    def _():
        blk_tok(per_w * NW + wid, 0)

    per_ws = nb_seq // NW
    blk_seq = make_block(stabT, seq_lin)
    lax.fori_loop(wid * per_ws, (wid + 1) * per_ws, blk_seq, 0)

    @pl.when(wid < nb_seq - per_ws * NW)
    def _():
        blk_seq(per_ws * NW + wid, 0)

    @pl.when(wid == 0)
    def _():
        tail(tok_tail, tok_lin, nb_tok)

    @pl.when(wid == 1)
    def _():
        tail(seq_tail, seq_lin, nb_seq)


_sc_transpose = functools.partial(
    pl.kernel,
    out_type=[
        jax.ShapeDtypeStruct((VOCAB * N_TOKEN_FIELDS * D,), jnp.float32),
        jax.ShapeDtypeStruct((VOCAB * N_SEQ_FIELDS * D,), jnp.float32),
    ],
    mesh=plsc.VectorSubcoreMesh(core_axis_name="c", subcore_axis_name="s"),
    compiler_params=pltpu.CompilerParams(use_tc_tiling_on_sc=True,
                                         needs_layout_passes=False),
    scratch_types=[
        pltpu.VMEM((16, 128), jnp.float32),
        pltpu.VMEM((2048,), jnp.float32),
        pltpu.SemaphoreType.DMA,
    ],
)(_tr_body)


def _sc_body(tok_idx_hbm, seq_idx_hbm, dense_hbm, tok_tab_hbm, seq_tab_hbm,
             out_hbm, ti_v, si_v, g_v, s_v, d_v, sem_t, sem_s):
    wid = lax.axis_index("s") * NC + lax.axis_index("c")

    def chunk(ci, _):
        base = wid * ROWS_PER_W + ci * NB
        pltpu.sync_copy(tok_idx_hbm.at[pl.ds(base * NBLK, NB * NBLK)], ti_v)
        pltpu.sync_copy(seq_idx_hbm.at[pl.ds(base * 100, NB * 100)], si_v)
        pltpu.sync_copy(dense_hbm.at[pl.ds(base * 2, NB * 2)], d_v)
        cps = []
        for j in range(NB * NBLK // TOK_SUB):
            cps.append(pltpu.async_copy(
                tok_tab_hbm.at[ti_v.at[pl.ds(j * TOK_SUB, TOK_SUB)]],
                g_v.at[pl.ds(j * TOK_SUB, TOK_SUB)], sem_t))
        for j in range(NB * 100 // SEQ_SUB):
            cps.append(pltpu.async_copy(
                seq_tab_hbm.at[si_v.at[pl.ds(j * SEQ_SUB, SEQ_SUB)]],
                s_v.at[pl.ds(j * SEQ_SUB, SEQ_SUB)], sem_s))
        for c in cps:
            c.wait()

        def body_b(b, _):
            b30 = b * NBLK
            for f in range(N_SEQ_FIELDS):
                r0 = b * 100 + f * SEQ_LEN
                a0 = s_v[r0 + 0]
                a1 = s_v[r0 + 1]
                a2 = s_v[r0 + 2]
                a3 = s_v[r0 + 3]
                for l in range(4, 48, 4):
                    a0 = a0 + s_v[r0 + l]
                    a1 = a1 + s_v[r0 + l + 1]
                    a2 = a2 + s_v[r0 + l + 2]
                    a3 = a3 + s_v[r0 + l + 3]
                a0 = a0 + s_v[r0 + 48]
                a1 = a1 + s_v[r0 + 49]
                tot = (a0 + a1) + (a2 + a3)
                g_v[b30 + 27 + f] = tot * jnp.float32(1.0 / SEQ_LEN)
            g_v[b30 + 26] = d_v[2 * b]
            g_v[b30 + 29] = d_v[2 * b + 1]
            return 0

        lax.fori_loop(0, NB, body_b, 0)
        pltpu.sync_copy(g_v, out_hbm.at[pl.ds(base * NBLK, NB * NBLK)])
        return 0

    lax.fori_loop(0, NCHUNK, chunk, 0)


_sc_gather = functools.partial(
    pl.kernel,
    out_type=jax.ShapeDtypeStruct((B * NBLK, D), jnp.float32),
    mesh=plsc.VectorSubcoreMesh(core_axis_name="c", subcore_axis_name="s"),
    compiler_params=pltpu.CompilerParams(use_tc_tiling_on_sc=False),
    scratch_types=[
        pltpu.VMEM((NB * NBLK,), jnp.int32),
        pltpu.VMEM((NB * 100,), jnp.int32),
        pltpu.VMEM((NB * NBLK, D), jnp.float32),
        pltpu.VMEM((NB * 100, D), jnp.float32),
        pltpu.VMEM((NB * 2, D), jnp.float32),
        pltpu.SemaphoreType.DMA,
        pltpu.SemaphoreType.DMA,
    ],
)(_sc_body)


def kernel(token_feature, float_feature, token_seq_feature, float_seq_feature,
           token_table, seq_table, num_W, num_b, numseq_W, numseq_b):
    tok_i = token_feature.astype(jnp.int32)
    seq_i = token_seq_feature.astype(jnp.int32).reshape(B, N_SEQ_FIELDS * SEQ_LEN)
    fsq2d = float_seq_feature.reshape(B, N_FLOAT_SEQ * SEQ_LEN)
    tok_idx, seq_idx, dense = _prep(tok_i, seq_i, float_feature, fsq2d,
                                    num_W, num_b, numseq_W, numseq_b)
    tok_lin, seq_lin = _sc_transpose(
        token_table.T, seq_table.T,
        token_table[(VOCAB * N_TOKEN_FIELDS) // 128 * 128:].reshape(-1),
        seq_table[(VOCAB * N_SEQ_FIELDS) // 128 * 128:].reshape(-1))
    out = _sc_gather(tok_idx.reshape(B * NBLK), seq_idx.reshape(B * 100),
                     dense.reshape(B * 2, D),
                     tok_lin.reshape(VOCAB * N_TOKEN_FIELDS, D),
                     seq_lin.reshape(VOCAB * N_SEQ_FIELDS, D))
    return out.reshape(B, NBLK * D)


# speedup vs baseline: 1.4095x; 1.4095x over previous
import functools

import jax
import jax.numpy as jnp
from jax import lax
from jax.experimental import pallas as pl
from jax.experimental.pallas import tpu as pltpu
from jax.experimental.pallas import tpu_sc as plsc

B = 16384
N_TOKEN_FIELDS = 26
N_SEQ_FIELDS = 2
N_FLOAT = 13
N_FLOAT_SEQ = 4
SEQ_LEN = 50
VOCAB = 100000
D = 16
NBLK = 30

NC = 2
NS = 16
NW = NC * NS
ROWS_PER_W = B // NW
NB = 32
NCHUNK = ROWS_PER_W // NB

TOK_SUB = 96
SEQ_SUB = 128


def _prep_body(tf_ref, tsf_ref, ff_ref, fsq_ref, numW_ref, numb_ref,
               nsW_ref, nsb_ref, tok_idx_ref, seq_idx_ref, dense_ref):
    bsz = tf_ref.shape[0]
    tf = tf_ref[...]
    f_ids = lax.broadcasted_iota(jnp.int32, (bsz, N_TOKEN_FIELDS), 1)
    tok_idx = tf + f_ids * VOCAB
    tok_idx_ref[...] = jnp.concatenate(
        [tok_idx, jnp.zeros((bsz, NBLK - N_TOKEN_FIELDS), jnp.int32)], axis=1)
    tsf = tsf_ref[...]
    s_ids = lax.broadcasted_iota(jnp.int32, (bsz, N_SEQ_FIELDS * SEQ_LEN), 1)
    seq_idx_ref[...] = tsf + (s_ids // SEQ_LEN) * VOCAB
    ff = ff_ref[...]
    num = jnp.broadcast_to(numb_ref[...][None, :], (bsz, D))
    for k in range(N_FLOAT):
        num = num + ff[:, k:k + 1] * numW_ref[k:k + 1, :]
    fsq = fsq_ref[...]
    ns = jnp.broadcast_to(nsb_ref[...][None, :], (bsz, D))
    for f in range(N_FLOAT_SEQ):
        seg = fsq[:, f * SEQ_LEN:(f + 1) * SEQ_LEN]
        cnt = jnp.sum((seg != 0.0).astype(jnp.float32), axis=1, keepdims=True)
        feat = jnp.sum(seg, axis=1, keepdims=True) / (cnt + 1e-08)
        ns = ns + feat * nsW_ref[f:f + 1, :]
    dense_ref[...] = jnp.concatenate([num, ns], axis=1)


def _prep(token_feature, token_seq2d, float_feature, float_seq2d,
          num_W, num_b, numseq_W, numseq_b):
    bsz = 2048
    grid = B // bsz
    row_blk = lambda w: pl.BlockSpec((bsz, w), lambda i: (i, 0))
    full = lambda shp: pl.BlockSpec(shp, lambda i: tuple(0 for _ in shp))
    return pl.pallas_call(
        _prep_body,
        grid=(grid,),
        in_specs=[
            row_blk(N_TOKEN_FIELDS),
            row_blk(N_SEQ_FIELDS * SEQ_LEN),
            row_blk(N_FLOAT),
            row_blk(N_FLOAT_SEQ * SEQ_LEN),
            full((N_FLOAT, D)),
            full((D,)),
            full((N_FLOAT_SEQ, D)),
            full((D,)),
        ],
        out_specs=[
            row_blk(NBLK),
            row_blk(N_SEQ_FIELDS * SEQ_LEN),
            row_blk(2 * D),
        ],
        out_shape=[
            jax.ShapeDtypeStruct((B, NBLK), jnp.int32),
            jax.ShapeDtypeStruct((B, N_SEQ_FIELDS * SEQ_LEN), jnp.int32),
            jax.ShapeDtypeStruct((B, 2 * D), jnp.float32),
        ],
    )(token_feature, token_seq2d, float_feature, float_seq2d,
      num_W, num_b, numseq_W, numseq_b)


def _tr_body(tabT, stabT, tok_tail, seq_tail, tok_lin, seq_lin,
             in0, in1, ou0, ou1, si0, si1, so0, so1):
    wid = lax.axis_index("s") * NC + lax.axis_index("c")
    rows16 = lax.iota(jnp.int32, 16)
    LW = 1024

    def tr_block(iv, ov):
        def sub(k, _):
            for l in range(128):
                lane = k * 128 + l
                lanes = jnp.full((16,), lane, jnp.int32)
                ov[pl.ds(lane * 16, 16)] = plsc.load_gather(iv, [rows16, lanes])
            return 0
        lax.fori_loop(0, LW // 128, sub, 0)

    def span(src, dst, lo, n_static, extra_pred, extra_g):
        ins = (in0, in1)
        ous = (ou0, ou1)
        sis = (si0, si1)
        sos = (so0, so1)

        def in_cp(g, b):
            return pltpu.async_copy(src.at[:, pl.ds(g * LW, LW)], ins[b], sis[b])

        def out_cp(g, b):
            return pltpu.async_copy(ous[b], dst.at[pl.ds(g * LW * 16, LW * 16)],
                                    sos[b])

        def in_wait(b):
            pltpu.make_async_copy(src.at[:, pl.ds(0, LW)], ins[b], sis[b]).wait()

        def out_wait(b):
            pltpu.make_async_copy(ous[b], dst.at[pl.ds(0, LW * 16)],
                                  sos[b]).wait()

        in_cp(lo, 0)
        nhalf = n_static // 2

        def body2(j, _):
            b0 = lo + 2 * j
            for b in range(2):
                in_wait(b)

                @pl.when(b0 + b + 1 < lo + n_static)
                def _():
                    in_cp(b0 + b + 1, 1 - b)

                @pl.when(2 * j + b >= 2)
                def _():
                    out_wait(b)
                tr_block(ins[b], ous[b])
                out_cp(b0 + b, b)
            return 0

        lax.fori_loop(0, nhalf, body2, 0)
        if n_static % 2:
            g = lo + n_static - 1
            in_wait(0)
            if n_static >= 3:
                out_wait(0)
            tr_block(in0, ou0)
            out_cp(g, 0)
        @pl.when(extra_pred)
        def _():
            in_cp(extra_g, 1 - (n_static % 2))
            in_wait(1 - (n_static % 2))
            if n_static >= 2:
                out_wait(1 - (n_static % 2))
            tr_block(ins[1 - (n_static % 2)], ous[1 - (n_static % 2)])
            out_cp(extra_g, 1 - (n_static % 2))
        out_wait(n_static % 2)
        out_wait(1 - (n_static % 2))

    nb_tok = (VOCAB * N_TOKEN_FIELDS) // LW
    nb_seq = (VOCAB * N_SEQ_FIELDS) // LW
    pw_t = nb_tok // NW
    pw_s = nb_seq // NW
    span(tabT, tok_lin, wid * pw_t, pw_t,
         wid < nb_tok - pw_t * NW, pw_t * NW + wid)
    span(stabT, seq_lin, wid * pw_s, pw_s,
         wid < nb_seq - pw_s * NW, pw_s * NW + wid)

    def tail(src_lin, dst, off):
        pltpu.sync_copy(src_lin, ou0.at[pl.ds(0, 1024)])
        pltpu.sync_copy(ou0.at[pl.ds(0, 1024)], dst.at[pl.ds(off, 1024)])

    @pl.when(wid == 0)
    def _():
        tail(tok_tail, tok_lin, nb_tok * LW * 16)

    @pl.when(wid == 1)
    def _():
        tail(seq_tail, seq_lin, nb_seq * LW * 16)


_sc_transpose = functools.partial(
    pl.kernel,
    out_type=[
        jax.ShapeDtypeStruct((VOCAB * N_TOKEN_FIELDS * D,), jnp.float32),
        jax.ShapeDtypeStruct((VOCAB * N_SEQ_FIELDS * D,), jnp.float32),
    ],
    mesh=plsc.VectorSubcoreMesh(core_axis_name="c", subcore_axis_name="s"),
    compiler_params=pltpu.CompilerParams(use_tc_tiling_on_sc=True,
                                         needs_layout_passes=False),
    scratch_types=[
        pltpu.VMEM((16, 1024), jnp.float32),
        pltpu.VMEM((16, 1024), jnp.float32),
        pltpu.VMEM((16384,), jnp.float32),
        pltpu.VMEM((16384,), jnp.float32),
        pltpu.SemaphoreType.DMA,
        pltpu.SemaphoreType.DMA,
        pltpu.SemaphoreType.DMA,
        pltpu.SemaphoreType.DMA,
    ],
)(_tr_body)


def _sc_body(tok_idx_hbm, seq_idx_hbm, dense_hbm, tok_tab_hbm, seq_tab_hbm,
             out_hbm, ti_v, si_v, g_v, s_v, d_v, sem_t, sem_s):
    wid = lax.axis_index("s") * NC + lax.axis_index("c")

    def chunk(ci, _):
        base = wid * ROWS_PER_W + ci * NB
        pltpu.sync_copy(tok_idx_hbm.at[pl.ds(base * NBLK, NB * NBLK)], ti_v)
        pltpu.sync_copy(seq_idx_hbm.at[pl.ds(base * 100, NB * 100)], si_v)
        pltpu.sync_copy(dense_hbm.at[pl.ds(base * 2, NB * 2)], d_v)
        cps = []
        for j in range(NB * NBLK // TOK_SUB):
            cps.append(pltpu.async_copy(
                tok_tab_hbm.at[ti_v.at[pl.ds(j * TOK_SUB, TOK_SUB)]],
                g_v.at[pl.ds(j * TOK_SUB, TOK_SUB)], sem_t))
        for j in range(NB * 100 // SEQ_SUB):
            cps.append(pltpu.async_copy(
                seq_tab_hbm.at[si_v.at[pl.ds(j * SEQ_SUB, SEQ_SUB)]],
                s_v.at[pl.ds(j * SEQ_SUB, SEQ_SUB)], sem_s))
        for c in cps:
            c.wait()

        def body_b(b, _):
            b30 = b * NBLK
            for f in range(N_SEQ_FIELDS):
                r0 = b * 100 + f * SEQ_LEN
                a0 = s_v[r0 + 0]
                a1 = s_v[r0 + 1]
                a2 = s_v[r0 + 2]
                a3 = s_v[r0 + 3]
                for l in range(4, 48, 4):
                    a0 = a0 + s_v[r0 + l]
                    a1 = a1 + s_v[r0 + l + 1]
                    a2 = a2 + s_v[r0 + l + 2]
                    a3 = a3 + s_v[r0 + l + 3]
                a0 = a0 + s_v[r0 + 48]
                a1 = a1 + s_v[r0 + 49]
                tot = (a0 + a1) + (a2 + a3)
                g_v[b30 + 27 + f] = tot * jnp.float32(1.0 / SEQ_LEN)
            g_v[b30 + 26] = d_v[2 * b]
            g_v[b30 + 29] = d_v[2 * b + 1]
            return 0

        lax.fori_loop(0, NB, body_b, 0)
        pltpu.sync_copy(g_v, out_hbm.at[pl.ds(base * NBLK, NB * NBLK)])
        return 0

    lax.fori_loop(0, NCHUNK, chunk, 0)


_sc_gather = functools.partial(
    pl.kernel,
    out_type=jax.ShapeDtypeStruct((B * NBLK, D), jnp.float32),
    mesh=plsc.VectorSubcoreMesh(core_axis_name="c", subcore_axis_name="s"),
    compiler_params=pltpu.CompilerParams(use_tc_tiling_on_sc=False),
    scratch_types=[
        pltpu.VMEM((NB * NBLK,), jnp.int32),
        pltpu.VMEM((NB * 100,), jnp.int32),
        pltpu.VMEM((NB * NBLK, D), jnp.float32),
        pltpu.VMEM((NB * 100, D), jnp.float32),
        pltpu.VMEM((NB * 2, D), jnp.float32),
        pltpu.SemaphoreType.DMA,
        pltpu.SemaphoreType.DMA,
    ],
)(_sc_body)


def kernel(token_feature, float_feature, token_seq_feature, float_seq_feature,
           token_table, seq_table, num_W, num_b, numseq_W, numseq_b):
    tok_i = token_feature.astype(jnp.int32)
    seq_i = token_seq_feature.astype(jnp.int32).reshape(B, N_SEQ_FIELDS * SEQ_LEN)
    fsq2d = float_seq_feature.reshape(B, N_FLOAT_SEQ * SEQ_LEN)
    tok_idx, seq_idx, dense = _prep(tok_i, seq_i, float_feature, fsq2d,
                                    num_W, num_b, numseq_W, numseq_b)
    tok_lin, seq_lin = _sc_transpose(
        token_table.T, seq_table.T,
        token_table[(VOCAB * N_TOKEN_FIELDS) // 128 * 128:].reshape(-1),
        seq_table[(VOCAB * N_SEQ_FIELDS) // 128 * 128:].reshape(-1))
    out = _sc_gather(tok_idx.reshape(B * NBLK), seq_idx.reshape(B * 100),
                     dense.reshape(B * 2, D),
                     tok_lin.reshape(VOCAB * N_TOKEN_FIELDS, D),
                     seq_lin.reshape(VOCAB * N_SEQ_FIELDS, D))
    return out.reshape(B, NBLK * D)


# speedup vs baseline: 2.8786x; 2.0423x over previous
import functools

import jax
import jax.numpy as jnp
from jax import lax
from jax.experimental import pallas as pl
from jax.experimental.pallas import tpu as pltpu
from jax.experimental.pallas import tpu_sc as plsc

B = 16384
N_TOKEN_FIELDS = 26
N_SEQ_FIELDS = 2
N_FLOAT = 13
N_FLOAT_SEQ = 4
SEQ_LEN = 50
VOCAB = 100000
D = 16
NBLK = 30

NC = 2
NS = 16
NW = NC * NS
ROWS_PER_W = B // NW
NB = 32
NCHUNK = ROWS_PER_W // NB

TOK_SUB = 96
SEQ_SUB = 128


def _prep_body(tf_ref, tsf_ref, ff_ref, fsq_ref, numW_ref, numb_ref,
               nsW_ref, nsb_ref, tok_idx_ref, seq_idx_ref, dense_ref):
    bsz = tf_ref.shape[0]
    tf = tf_ref[...]
    f_ids = lax.broadcasted_iota(jnp.int32, (bsz, N_TOKEN_FIELDS), 1)
    tok_idx = tf + f_ids * VOCAB
    tok_idx_ref[...] = jnp.concatenate(
        [tok_idx, jnp.zeros((bsz, NBLK - N_TOKEN_FIELDS), jnp.int32)], axis=1)
    tsf = tsf_ref[...]
    s_ids = lax.broadcasted_iota(jnp.int32, (bsz, N_SEQ_FIELDS * SEQ_LEN), 1)
    seq_idx_ref[...] = tsf + (s_ids // SEQ_LEN) * VOCAB
    ff = ff_ref[...]
    num = jnp.broadcast_to(numb_ref[...][None, :], (bsz, D))
    for k in range(N_FLOAT):
        num = num + ff[:, k:k + 1] * numW_ref[k:k + 1, :]
    fsq = fsq_ref[...]
    ns = jnp.broadcast_to(nsb_ref[...][None, :], (bsz, D))
    for f in range(N_FLOAT_SEQ):
        seg = fsq[:, f * SEQ_LEN:(f + 1) * SEQ_LEN]
        cnt = jnp.sum((seg != 0.0).astype(jnp.float32), axis=1, keepdims=True)
        feat = jnp.sum(seg, axis=1, keepdims=True) / (cnt + 1e-08)
        ns = ns + feat * nsW_ref[f:f + 1, :]
    dense_ref[...] = jnp.concatenate([num, ns], axis=1)


def _prep(token_feature, token_seq2d, float_feature, float_seq2d,
          num_W, num_b, numseq_W, numseq_b):
    bsz = 2048
    grid = B // bsz
    row_blk = lambda w: pl.BlockSpec((bsz, w), lambda i: (i, 0))
    full = lambda shp: pl.BlockSpec(shp, lambda i: tuple(0 for _ in shp))
    return pl.pallas_call(
        _prep_body,
        grid=(grid,),
        in_specs=[
            row_blk(N_TOKEN_FIELDS),
            row_blk(N_SEQ_FIELDS * SEQ_LEN),
            row_blk(N_FLOAT),
            row_blk(N_FLOAT_SEQ * SEQ_LEN),
            full((N_FLOAT, D)),
            full((D,)),
            full((N_FLOAT_SEQ, D)),
            full((D,)),
        ],
        out_specs=[
            row_blk(NBLK),
            row_blk(N_SEQ_FIELDS * SEQ_LEN),
            row_blk(2 * D),
        ],
        out_shape=[
            jax.ShapeDtypeStruct((B, NBLK), jnp.int32),
            jax.ShapeDtypeStruct((B, N_SEQ_FIELDS * SEQ_LEN), jnp.int32),
            jax.ShapeDtypeStruct((B, 2 * D), jnp.float32),
        ],
    )(token_feature, token_seq2d, float_feature, float_seq2d,
      num_W, num_b, numseq_W, numseq_b)


def _tr_body(tabT, stabT, tok_tail, seq_tail, tok_lin, seq_lin,
             in0, in1, ou0, ou1, si0, si1, so0, so1):
    wid = lax.axis_index("s") * NC + lax.axis_index("c")
    LW = 1024
    scat_idx = [lax.iota(jnp.int32, 16) * 16 + s for s in range(16)]

    def tr_block(iv, ov):
        @plsc.parallel_loop(0, LW // 16, 1, unroll=4)
        def sub(t):
            t16 = t * 16
            ov_sl = ov.at[pl.ds(t16 * 16, 256)]
            for s in range(16):
                plsc.store_scatter(ov_sl, [scat_idx[s]],
                                   iv[s, pl.ds(t16, 16)])

    def span(src, dst, lo, n_static, extra_pred, extra_g):
        ins = (in0, in1)
        ous = (ou0, ou1)
        sis = (si0, si1)
        sos = (so0, so1)

        def in_cp(g, b):
            return pltpu.async_copy(src.at[:, pl.ds(g * LW, LW)], ins[b], sis[b])

        def out_cp(g, b):
            return pltpu.async_copy(ous[b], dst.at[pl.ds(g * LW * 16, LW * 16)],
                                    sos[b])

        def in_wait(b):
            pltpu.make_async_copy(src.at[:, pl.ds(0, LW)], ins[b], sis[b]).wait()

        def out_wait(b):
            pltpu.make_async_copy(ous[b], dst.at[pl.ds(0, LW * 16)],
                                  sos[b]).wait()

        in_cp(lo, 0)
        nhalf = n_static // 2

        def body2(j, _):
            b0 = lo + 2 * j
            for b in range(2):
                in_wait(b)

                @pl.when(b0 + b + 1 < lo + n_static)
                def _():
                    in_cp(b0 + b + 1, 1 - b)

                @pl.when(2 * j + b >= 2)
                def _():
                    out_wait(b)
                tr_block(ins[b], ous[b])
                out_cp(b0 + b, b)
            return 0

        lax.fori_loop(0, nhalf, body2, 0)
        if n_static % 2:
            g = lo + n_static - 1
            in_wait(0)
            if n_static >= 3:
                out_wait(0)
            tr_block(in0, ou0)
            out_cp(g, 0)
        @pl.when(extra_pred)
        def _():
            in_cp(extra_g, 1 - (n_static % 2))
            in_wait(1 - (n_static % 2))
            if n_static >= 2:
                out_wait(1 - (n_static % 2))
            tr_block(ins[1 - (n_static % 2)], ous[1 - (n_static % 2)])
            out_cp(extra_g, 1 - (n_static % 2))
        out_wait(n_static % 2)
        out_wait(1 - (n_static % 2))

    nb_tok = (VOCAB * N_TOKEN_FIELDS) // LW
    nb_seq = (VOCAB * N_SEQ_FIELDS) // LW
    pw_t = nb_tok // NW
    pw_s = nb_seq // NW
    span(tabT, tok_lin, wid * pw_t, pw_t,
         wid < nb_tok - pw_t * NW, pw_t * NW + wid)
    span(stabT, seq_lin, wid * pw_s, pw_s,
         wid < nb_seq - pw_s * NW, pw_s * NW + wid)

    def tail(src_lin, dst, off):
        pltpu.sync_copy(src_lin, ou0.at[pl.ds(0, 1024)])
        pltpu.sync_copy(ou0.at[pl.ds(0, 1024)], dst.at[pl.ds(off, 1024)])

    @pl.when(wid == 0)
    def _():
        tail(tok_tail, tok_lin, nb_tok * LW * 16)

    @pl.when(wid == 1)
    def _():
        tail(seq_tail, seq_lin, nb_seq * LW * 16)


_sc_transpose = functools.partial(
    pl.kernel,
    out_type=[
        jax.ShapeDtypeStruct((VOCAB * N_TOKEN_FIELDS * D,), jnp.float32),
        jax.ShapeDtypeStruct((VOCAB * N_SEQ_FIELDS * D,), jnp.float32),
    ],
    mesh=plsc.VectorSubcoreMesh(core_axis_name="c", subcore_axis_name="s"),
    compiler_params=pltpu.CompilerParams(use_tc_tiling_on_sc=True,
                                         needs_layout_passes=False),
    scratch_types=[
        pltpu.VMEM((16, 1024), jnp.float32),
        pltpu.VMEM((16, 1024), jnp.float32),
        pltpu.VMEM((16384,), jnp.float32),
        pltpu.VMEM((16384,), jnp.float32),
        pltpu.SemaphoreType.DMA,
        pltpu.SemaphoreType.DMA,
        pltpu.SemaphoreType.DMA,
        pltpu.SemaphoreType.DMA,
    ],
)(_tr_body)


def _sc_body(tok_idx_hbm, seq_idx_hbm, dense_hbm, tok_tab_hbm, seq_tab_hbm,
             out_hbm, ti_v, si_v, g_v, s_v, d_v, sem_t, sem_s):
    wid = lax.axis_index("s") * NC + lax.axis_index("c")

    def chunk(ci, _):
        base = wid * ROWS_PER_W + ci * NB
        pltpu.sync_copy(tok_idx_hbm.at[pl.ds(base * NBLK, NB * NBLK)], ti_v)
        pltpu.sync_copy(seq_idx_hbm.at[pl.ds(base * 100, NB * 100)], si_v)
        pltpu.sync_copy(dense_hbm.at[pl.ds(base * 2, NB * 2)], d_v)
        cps = []
        for j in range(NB * NBLK // TOK_SUB):
            cps.append(pltpu.async_copy(
                tok_tab_hbm.at[ti_v.at[pl.ds(j * TOK_SUB, TOK_SUB)]],
                g_v.at[pl.ds(j * TOK_SUB, TOK_SUB)], sem_t))
        for j in range(NB * 100 // SEQ_SUB):
            cps.append(pltpu.async_copy(
                seq_tab_hbm.at[si_v.at[pl.ds(j * SEQ_SUB, SEQ_SUB)]],
                s_v.at[pl.ds(j * SEQ_SUB, SEQ_SUB)], sem_s))
        for c in cps:
            c.wait()

        def body_b(b, _):
            b30 = b * NBLK
            for f in range(N_SEQ_FIELDS):
                r0 = b * 100 + f * SEQ_LEN
                a0 = s_v[r0 + 0]
                a1 = s_v[r0 + 1]
                a2 = s_v[r0 + 2]
                a3 = s_v[r0 + 3]
                for l in range(4, 48, 4):
                    a0 = a0 + s_v[r0 + l]
                    a1 = a1 + s_v[r0 + l + 1]
                    a2 = a2 + s_v[r0 + l + 2]
                    a3 = a3 + s_v[r0 + l + 3]
                a0 = a0 + s_v[r0 + 48]
                a1 = a1 + s_v[r0 + 49]
                tot = (a0 + a1) + (a2 + a3)
                g_v[b30 + 27 + f] = tot * jnp.float32(1.0 / SEQ_LEN)
            g_v[b30 + 26] = d_v[2 * b]
            g_v[b30 + 29] = d_v[2 * b + 1]
            return 0

        lax.fori_loop(0, NB, body_b, 0)
        pltpu.sync_copy(g_v, out_hbm.at[pl.ds(base * NBLK, NB * NBLK)])
        return 0

    lax.fori_loop(0, NCHUNK, chunk, 0)


_sc_gather = functools.partial(
    pl.kernel,
    out_type=jax.ShapeDtypeStruct((B * NBLK, D), jnp.float32),
    mesh=plsc.VectorSubcoreMesh(core_axis_name="c", subcore_axis_name="s"),
    compiler_params=pltpu.CompilerParams(use_tc_tiling_on_sc=False),
    scratch_types=[
        pltpu.VMEM((NB * NBLK,), jnp.int32),
        pltpu.VMEM((NB * 100,), jnp.int32),
        pltpu.VMEM((NB * NBLK, D), jnp.float32),
        pltpu.VMEM((NB * 100, D), jnp.float32),
        pltpu.VMEM((NB * 2, D), jnp.float32),
        pltpu.SemaphoreType.DMA,
        pltpu.SemaphoreType.DMA,
    ],
)(_sc_body)


def kernel(token_feature, float_feature, token_seq_feature, float_seq_feature,
           token_table, seq_table, num_W, num_b, numseq_W, numseq_b):
    tok_i = token_feature.astype(jnp.int32)
    seq_i = token_seq_feature.astype(jnp.int32).reshape(B, N_SEQ_FIELDS * SEQ_LEN)
    fsq2d = float_seq_feature.reshape(B, N_FLOAT_SEQ * SEQ_LEN)
    tok_idx, seq_idx, dense = _prep(tok_i, seq_i, float_feature, fsq2d,
                                    num_W, num_b, numseq_W, numseq_b)
    tok_lin, seq_lin = _sc_transpose(
        token_table.T, seq_table.T,
        token_table[(VOCAB * N_TOKEN_FIELDS) // 128 * 128:].reshape(-1),
        seq_table[(VOCAB * N_SEQ_FIELDS) // 128 * 128:].reshape(-1))
    out = _sc_gather(tok_idx.reshape(B * NBLK), seq_idx.reshape(B * 100),
                     dense.reshape(B * 2, D),
                     tok_lin.reshape(VOCAB * N_TOKEN_FIELDS, D),
                     seq_lin.reshape(VOCAB * N_SEQ_FIELDS, D))
    return out.reshape(B, NBLK * D)


# speedup vs baseline: 3.1528x; 1.0953x over previous
import functools

import jax
import jax.numpy as jnp
from jax import lax
from jax.experimental import pallas as pl
from jax.experimental.pallas import tpu as pltpu
from jax.experimental.pallas import tpu_sc as plsc

B = 16384
N_TOKEN_FIELDS = 26
N_SEQ_FIELDS = 2
N_FLOAT = 13
N_FLOAT_SEQ = 4
SEQ_LEN = 50
VOCAB = 100000
D = 16
NBLK = 30

NC = 2
NS = 16
NW = NC * NS
ROWS_PER_W = B // NW
NB = 32
NCHUNK = ROWS_PER_W // NB

TOK_SUB = 96
SEQ_SUB = 128


def _prep_body(tf_ref, tsf_ref, ff_ref, fsq_ref, numW_ref, numb_ref,
               nsW_ref, nsb_ref, tok_idx_ref, seq_idx_ref, dense_ref):
    bsz = tf_ref.shape[0]
    tf = tf_ref[...]
    f_ids = lax.broadcasted_iota(jnp.int32, (bsz, N_TOKEN_FIELDS), 1)
    tok_idx = tf + f_ids * VOCAB
    tok_idx_ref[...] = jnp.concatenate(
        [tok_idx, jnp.zeros((bsz, NBLK - N_TOKEN_FIELDS), jnp.int32)], axis=1)
    tsf = tsf_ref[...]
    s_ids = lax.broadcasted_iota(jnp.int32, (bsz, N_SEQ_FIELDS * SEQ_LEN), 1)
    seq_idx_ref[...] = tsf + (s_ids // SEQ_LEN) * VOCAB
    ff = ff_ref[...]
    num = jnp.broadcast_to(numb_ref[...][None, :], (bsz, D))
    for k in range(N_FLOAT):
        num = num + ff[:, k:k + 1] * numW_ref[k:k + 1, :]
    fsq = fsq_ref[...]
    ns = jnp.broadcast_to(nsb_ref[...][None, :], (bsz, D))
    for f in range(N_FLOAT_SEQ):
        seg = fsq[:, f * SEQ_LEN:(f + 1) * SEQ_LEN]
        cnt = jnp.sum((seg != 0.0).astype(jnp.float32), axis=1, keepdims=True)
        feat = jnp.sum(seg, axis=1, keepdims=True) / (cnt + 1e-08)
        ns = ns + feat * nsW_ref[f:f + 1, :]
    dense_ref[...] = jnp.concatenate([num, ns], axis=1)


def _prep(token_feature, token_seq2d, float_feature, float_seq2d,
          num_W, num_b, numseq_W, numseq_b):
    bsz = 2048
    grid = B // bsz
    row_blk = lambda w: pl.BlockSpec((bsz, w), lambda i: (i, 0))
    full = lambda shp: pl.BlockSpec(shp, lambda i: tuple(0 for _ in shp))
    return pl.pallas_call(
        _prep_body,
        grid=(grid,),
        in_specs=[
            row_blk(N_TOKEN_FIELDS),
            row_blk(N_SEQ_FIELDS * SEQ_LEN),
            row_blk(N_FLOAT),
            row_blk(N_FLOAT_SEQ * SEQ_LEN),
            full((N_FLOAT, D)),
            full((D,)),
            full((N_FLOAT_SEQ, D)),
            full((D,)),
        ],
        out_specs=[
            row_blk(NBLK),
            row_blk(N_SEQ_FIELDS * SEQ_LEN),
            row_blk(2 * D),
        ],
        out_shape=[
            jax.ShapeDtypeStruct((B, NBLK), jnp.int32),
            jax.ShapeDtypeStruct((B, N_SEQ_FIELDS * SEQ_LEN), jnp.int32),
            jax.ShapeDtypeStruct((B, 2 * D), jnp.float32),
        ],
    )(token_feature, token_seq2d, float_feature, float_seq2d,
      num_W, num_b, numseq_W, numseq_b)


def _tr_body(tabT, stabT, tok_tail, seq_tail, tok_lin, seq_lin,
             in0, in1, ou0, ou1, si0, si1, so0, so1):
    wid = lax.axis_index("s") * NC + lax.axis_index("c")
    LW = 1024
    scat_idx = [lax.iota(jnp.int32, 16) * 16 + s for s in range(16)]

    def tr_block(iv, ov):
        def sub(k, _):
            for tt in range(4):
                t16 = (k * 4 + tt) * 32
                vals = [iv[s, pl.ds(t16 + 16 * h, 16)]
                        for h in range(2) for s in range(16)]
                for h in range(2):
                    ov_sl = ov.at[pl.ds((t16 + 16 * h) * 16, 256)]
                    for s in range(16):
                        plsc.store_scatter(ov_sl, [scat_idx[s]],
                                           vals[h * 16 + s])
            return 0
        lax.fori_loop(0, LW // 128, sub, 0)

    def span(src, dst, lo, n_static, extra_pred, extra_g):
        ins = (in0, in1)
        ous = (ou0, ou1)
        sis = (si0, si1)
        sos = (so0, so1)

        def in_cp(g, b):
            return pltpu.async_copy(src.at[:, pl.ds(g * LW, LW)], ins[b], sis[b])

        def out_cp(g, b):
            return pltpu.async_copy(ous[b], dst.at[pl.ds(g * LW * 16, LW * 16)],
                                    sos[b])

        def in_wait(b):
            pltpu.make_async_copy(src.at[:, pl.ds(0, LW)], ins[b], sis[b]).wait()

        def out_wait(b):
            pltpu.make_async_copy(ous[b], dst.at[pl.ds(0, LW * 16)],
                                  sos[b]).wait()

        in_cp(lo, 0)
        nhalf = n_static // 2

        def body2(j, _):
            b0 = lo + 2 * j
            for b in range(2):
                in_wait(b)

                @pl.when(b0 + b + 1 < lo + n_static)
                def _():
                    in_cp(b0 + b + 1, 1 - b)

                @pl.when(2 * j + b >= 2)
                def _():
                    out_wait(b)
                tr_block(ins[b], ous[b])
                out_cp(b0 + b, b)
            return 0

        lax.fori_loop(0, nhalf, body2, 0)
        if n_static % 2:
            g = lo + n_static - 1
            in_wait(0)
            if n_static >= 3:
                out_wait(0)
            tr_block(in0, ou0)
            out_cp(g, 0)
        @pl.when(extra_pred)
        def _():
            in_cp(extra_g, 1 - (n_static % 2))
            in_wait(1 - (n_static % 2))
            if n_static >= 2:
                out_wait(1 - (n_static % 2))
            tr_block(ins[1 - (n_static % 2)], ous[1 - (n_static % 2)])
            out_cp(extra_g, 1 - (n_static % 2))
        out_wait(n_static % 2)
        out_wait(1 - (n_static % 2))

    nb_tok = (VOCAB * N_TOKEN_FIELDS) // LW
    nb_seq = (VOCAB * N_SEQ_FIELDS) // LW
    pw_t = nb_tok // NW
    pw_s = nb_seq // NW
    span(tabT, tok_lin, wid * pw_t, pw_t,
         wid < nb_tok - pw_t * NW, pw_t * NW + wid)
    span(stabT, seq_lin, wid * pw_s, pw_s,
         wid < nb_seq - pw_s * NW, pw_s * NW + wid)

    def tail(src_lin, dst, off):
        pltpu.sync_copy(src_lin, ou0.at[pl.ds(0, 1024)])
        pltpu.sync_copy(ou0.at[pl.ds(0, 1024)], dst.at[pl.ds(off, 1024)])

    @pl.when(wid == 0)
    def _():
        tail(tok_tail, tok_lin, nb_tok * LW * 16)

    @pl.when(wid == 1)
    def _():
        tail(seq_tail, seq_lin, nb_seq * LW * 16)


_sc_transpose = functools.partial(
    pl.kernel,
    out_type=[
        jax.ShapeDtypeStruct((VOCAB * N_TOKEN_FIELDS * D,), jnp.float32),
        jax.ShapeDtypeStruct((VOCAB * N_SEQ_FIELDS * D,), jnp.float32),
    ],
    mesh=plsc.VectorSubcoreMesh(core_axis_name="c", subcore_axis_name="s"),
    compiler_params=pltpu.CompilerParams(use_tc_tiling_on_sc=True,
                                         needs_layout_passes=False),
    scratch_types=[
        pltpu.VMEM((16, 1024), jnp.float32),
        pltpu.VMEM((16, 1024), jnp.float32),
        pltpu.VMEM((16384,), jnp.float32),
        pltpu.VMEM((16384,), jnp.float32),
        pltpu.SemaphoreType.DMA,
        pltpu.SemaphoreType.DMA,
        pltpu.SemaphoreType.DMA,
        pltpu.SemaphoreType.DMA,
    ],
)(_tr_body)


def _sc_body(tok_idx_hbm, seq_idx_hbm, dense_hbm, tok_tab_hbm, seq_tab_hbm,
             out_hbm, ti_v, si_v, g_v, s_v, d_v, sem_t, sem_s):
    wid = lax.axis_index("s") * NC + lax.axis_index("c")

    def chunk(ci, _):
        base = wid * ROWS_PER_W + ci * NB
        pltpu.sync_copy(tok_idx_hbm.at[pl.ds(base * NBLK, NB * NBLK)], ti_v)
        pltpu.sync_copy(seq_idx_hbm.at[pl.ds(base * 100, NB * 100)], si_v)
        pltpu.sync_copy(dense_hbm.at[pl.ds(base * 2, NB * 2)], d_v)
        cps = []
        for j in range(NB * NBLK // TOK_SUB):
            cps.append(pltpu.async_copy(
                tok_tab_hbm.at[ti_v.at[pl.ds(j * TOK_SUB, TOK_SUB)]],
                g_v.at[pl.ds(j * TOK_SUB, TOK_SUB)], sem_t))
        for j in range(NB * 100 // SEQ_SUB):
            cps.append(pltpu.async_copy(
                seq_tab_hbm.at[si_v.at[pl.ds(j * SEQ_SUB, SEQ_SUB)]],
                s_v.at[pl.ds(j * SEQ_SUB, SEQ_SUB)], sem_s))
        for c in cps:
            c.wait()

        def body_b(b, _):
            b30 = b * NBLK
            for f in range(N_SEQ_FIELDS):
                r0 = b * 100 + f * SEQ_LEN
                a0 = s_v[r0 + 0]
                a1 = s_v[r0 + 1]
                a2 = s_v[r0 + 2]
                a3 = s_v[r0 + 3]
                for l in range(4, 48, 4):
                    a0 = a0 + s_v[r0 + l]
                    a1 = a1 + s_v[r0 + l + 1]
                    a2 = a2 + s_v[r0 + l + 2]
                    a3 = a3 + s_v[r0 + l + 3]
                a0 = a0 + s_v[r0 + 48]
                a1 = a1 + s_v[r0 + 49]
                tot = (a0 + a1) + (a2 + a3)
                g_v[b30 + 27 + f] = tot * jnp.float32(1.0 / SEQ_LEN)
            g_v[b30 + 26] = d_v[2 * b]
            g_v[b30 + 29] = d_v[2 * b + 1]
            return 0

        lax.fori_loop(0, NB, body_b, 0)
        pltpu.sync_copy(g_v, out_hbm.at[pl.ds(base * NBLK, NB * NBLK)])
        return 0

    lax.fori_loop(0, NCHUNK, chunk, 0)


_sc_gather = functools.partial(
    pl.kernel,
    out_type=jax.ShapeDtypeStruct((B * NBLK, D), jnp.float32),
    mesh=plsc.VectorSubcoreMesh(core_axis_name="c", subcore_axis_name="s"),
    compiler_params=pltpu.CompilerParams(use_tc_tiling_on_sc=False),
    scratch_types=[
        pltpu.VMEM((NB * NBLK,), jnp.int32),
        pltpu.VMEM((NB * 100,), jnp.int32),
        pltpu.VMEM((NB * NBLK, D), jnp.float32),
        pltpu.VMEM((NB * 100, D), jnp.float32),
        pltpu.VMEM((NB * 2, D), jnp.float32),
        pltpu.SemaphoreType.DMA,
        pltpu.SemaphoreType.DMA,
    ],
)(_sc_body)


def kernel(token_feature, float_feature, token_seq_feature, float_seq_feature,
           token_table, seq_table, num_W, num_b, numseq_W, numseq_b):
    tok_i = token_feature.astype(jnp.int32)
    seq_i = token_seq_feature.astype(jnp.int32).reshape(B, N_SEQ_FIELDS * SEQ_LEN)
    fsq2d = float_seq_feature.reshape(B, N_FLOAT_SEQ * SEQ_LEN)
    tok_idx, seq_idx, dense = _prep(tok_i, seq_i, float_feature, fsq2d,
                                    num_W, num_b, numseq_W, numseq_b)
    tok_lin, seq_lin = _sc_transpose(
        token_table.T, seq_table.T,
        token_table[(VOCAB * N_TOKEN_FIELDS) // 128 * 128:].reshape(-1),
        seq_table[(VOCAB * N_SEQ_FIELDS) // 128 * 128:].reshape(-1))
    out = _sc_gather(tok_idx.reshape(B * NBLK), seq_idx.reshape(B * 100),
                     dense.reshape(B * 2, D),
                     tok_lin.reshape(VOCAB * N_TOKEN_FIELDS, D),
                     seq_lin.reshape(VOCAB * N_SEQ_FIELDS, D))
    return out.reshape(B, NBLK * D)
